# Initial kernel scaffold; baseline (speedup 1.0000x reference)
import functools

import jax
import jax.numpy as jnp
from jax import lax
from jax.experimental import pallas as pl
from jax.experimental.pallas import tpu as pltpu
from jax.experimental.pallas import tpu_sc as plsc

NC = 2
NS = 16
NW = NC * NS
LANES = 16

N = 10000
NPAD = 10240
D = 128
BLK = 128
E = 320000
BPW = -(-E // (NW * BLK))
EPAD = NW * BPW * BLK

ROWS_PER_TILE = NPAD // NS
F32 = jnp.float32


def _mesh():
    return plsc.VectorSubcoreMesh(
        core_axis_name="c", subcore_axis_name="s", num_cores=NC, num_subcores=NS)


def _bcast_lane(wv, l):
    idx = jnp.full((LANES, 1), l, jnp.int32)
    dn = lax.GatherDimensionNumbers(
        offset_dims=(), collapsed_slice_dims=(0,), start_index_map=(0,))
    return lax.gather(wv, idx, dn, slice_sizes=(1,),
                      mode=lax.GatherScatterMode.PROMISE_IN_BOUNDS)


def _sc_deg_body(col_hbm, ew_hbm, out_hbm, col_v, ew_v, deg_t):
    cid = lax.axis_index("c")
    sid = lax.axis_index("s")
    w = cid * NS + sid
    pltpu.sync_copy(col_hbm.at[pl.ds(w * BPW, BPW)], col_v)
    pltpu.sync_copy(ew_hbm.at[pl.ds(w * BPW, BPW)], ew_v)

    zero = jnp.zeros((LANES,), F32)

    @pl.loop(0, NPAD // LANES)
    def _(i):
        deg_t[pl.ds(i * LANES, LANES)] = zero

    @pl.loop(0, BPW)
    def _(r):
        for c8 in range(BLK // LANES):
            cv = col_v[r, pl.ds(c8 * LANES, LANES)]
            wv = ew_v[r, pl.ds(c8 * LANES, LANES)]
            plsc.addupdate_scatter(deg_t, [cv], wv)

    pltpu.sync_copy(deg_t, out_hbm.at[w])


def _sc_deg(col_p, ew_p):
    k = pl.kernel(
        _sc_deg_body,
        out_type=jax.ShapeDtypeStruct((NW, NPAD), F32),
        mesh=_mesh(),
        scratch_types=[
            pltpu.VMEM((BPW, BLK), jnp.int32),
            pltpu.VMEM((BPW, BLK), F32),
            pltpu.VMEM((NPAD,), F32),
        ],
    )
    return k(col_p, ew_p)


def _sc_conv_body(g_hbm, row_hbm, col_hbm, ew_hbm, out_hbm,
                  row_v, col_v, ew_v, gbuf, acc):
    cid = lax.axis_index("c")
    sid = lax.axis_index("s")
    w = cid * NS + sid

    pltpu.sync_copy(row_hbm.at[pl.ds(w * BPW, BPW)], row_v)
    pltpu.sync_copy(col_hbm.at[pl.ds(w * BPW, BPW)], col_v)
    pltpu.sync_copy(ew_hbm.at[pl.ds(w * BPW, BPW)], ew_v)

    zero = jnp.zeros((LANES,), F32)

    @pl.loop(0, BLK)
    def _(r):
        for c8 in range(D // LANES):
            gbuf[r, pl.ds(c8 * LANES, LANES)] = zero

    for k in range(ROWS_PER_TILE // BLK):
        pltpu.sync_copy(gbuf, acc.at[pl.ds(sid * ROWS_PER_TILE + k * BLK, BLK)])
    plsc.subcore_barrier()

    @pl.loop(0, BPW)
    def _(j):
        pltpu.sync_copy(g_hbm.at[row_v.at[j]], gbuf)

        @pl.loop(0, BLK // LANES)
        def _(gi):
            wv = ew_v[j, pl.ds(gi * LANES, LANES)]
            for l in range(LANES):
                bc = _bcast_lane(wv, l)
                e = gi * LANES + l
                for d2 in range(D // LANES):
                    sl = pl.ds(d2 * LANES, LANES)
                    gbuf[e, sl] = gbuf[e, sl] * bc

        pltpu.sync_copy(gbuf, acc.at[col_v.at[j]], add=True)

    plsc.subcore_barrier()
    pltpu.sync_copy(acc.at[pl.ds(sid * ROWS_PER_TILE, ROWS_PER_TILE)],
                    out_hbm.at[cid, pl.ds(sid * ROWS_PER_TILE, ROWS_PER_TILE)])


def _sc_conv(g, row_p, col_p, ew_p):
    k = pl.kernel(
        _sc_conv_body,
        out_type=jax.ShapeDtypeStruct((NC, NPAD, D), F32),
        mesh=_mesh(),
        scratch_types=[
            pltpu.VMEM((BPW, BLK), jnp.int32),
            pltpu.VMEM((BPW, BLK), jnp.int32),
            pltpu.VMEM((BPW, BLK), F32),
            pltpu.VMEM((BLK, D), F32),
            pltpu.VMEM_SHARED((NPAD, D), F32),
        ],
    )
    return k(g, row_p, col_p, ew_p)


RB = 1024
GRID = NPAD // RB


def _tc_lin1_body(x_ref, win_ref, bin_ref, w1_ref, degp_ref,
                  hin_ref, t1_ref, g1_ref, dinv_ref):
    xin = x_ref[...]
    hin = jnp.maximum(xin @ win_ref[...] + bin_ref[...], 0.0)
    deg = jnp.sum(degp_ref[...], axis=0) + 1.0
    dinv = lax.rsqrt(deg)[:, None]
    t1 = hin @ w1_ref[...]
    hin_ref[...] = hin
    t1_ref[...] = t1
    g1_ref[...] = t1 * dinv
    dinv_ref[...] = dinv


def _tc_lin1(x_p, W_in, b_in, W1, deg_parts):
    wspec = pl.BlockSpec((D, D), lambda i: (0, 0))
    bspec = pl.BlockSpec((1, D), lambda i: (0, 0))
    rspec = pl.BlockSpec((RB, D), lambda i: (i, 0))
    return pl.pallas_call(
        _tc_lin1_body,
        grid=(GRID,),
        in_specs=[rspec, wspec, bspec, wspec,
                  pl.BlockSpec((NW, RB), lambda i: (0, i))],
        out_specs=[rspec, rspec, rspec, pl.BlockSpec((RB, 1), lambda i: (i, 0))],
        out_shape=[
            jax.ShapeDtypeStruct((NPAD, D), F32),
            jax.ShapeDtypeStruct((NPAD, D), F32),
            jax.ShapeDtypeStruct((NPAD, D), F32),
            jax.ShapeDtypeStruct((NPAD, 1), F32),
        ],
    )(x_p, W_in, b_in, W1, deg_parts)


def _tc_comb_body(sa_ref, sb_ref, t_ref, dinv_ref, b_ref, w2_ref,
                  t2_ref, g2_ref):
    dinv = dinv_ref[...]
    t = t_ref[...]
    h = jnp.maximum(dinv * (sa_ref[...] + sb_ref[...]) + t * (dinv * dinv)
                    + b_ref[...], 0.0)
    t2 = h @ w2_ref[...]
    t2_ref[...] = t2
    g2_ref[...] = t2 * dinv


def _tc_comb(sa, sb, t1, dinv, b1, W2):
    wspec = pl.BlockSpec((D, D), lambda i: (0, 0))
    bspec = pl.BlockSpec((1, D), lambda i: (0, 0))
    rspec = pl.BlockSpec((RB, D), lambda i: (i, 0))
    dspec = pl.BlockSpec((RB, 1), lambda i: (i, 0))
    return pl.pallas_call(
        _tc_comb_body,
        grid=(GRID,),
        in_specs=[rspec, rspec, rspec, dspec, bspec, wspec],
        out_specs=[rspec, rspec],
        out_shape=[
            jax.ShapeDtypeStruct((NPAD, D), F32),
            jax.ShapeDtypeStruct((NPAD, D), F32),
        ],
    )(sa, sb, t1, dinv, b1, W2)


def _tc_final_body(sa_ref, sb_ref, t2_ref, dinv_ref, b2_ref, hin_ref,
                   wm1a_ref, wm1b_ref, bm1_ref, wm2_ref, bm2_ref, out_ref):
    dinv = dinv_ref[...]
    t2 = t2_ref[...]
    h2 = jnp.maximum(dinv * (sa_ref[...] + sb_ref[...]) + t2 * (dinv * dinv)
                     + b2_ref[...], 0.0)
    hidden = jnp.maximum(
        hin_ref[...] @ wm1a_ref[...] + h2 @ wm1b_ref[...] + bm1_ref[...], 0.0)
    out_ref[...] = hidden @ wm2_ref[...] + bm2_ref[...]


def _tc_final(sa, sb, t2, dinv, b2, h_in, Wm1a, Wm1b, bm1, Wm2, bm2):
    wspec = pl.BlockSpec((D, D), lambda i: (0, 0))
    bspec = pl.BlockSpec((1, D), lambda i: (0, 0))
    rspec = pl.BlockSpec((RB, D), lambda i: (i, 0))
    dspec = pl.BlockSpec((RB, 1), lambda i: (i, 0))
    return pl.pallas_call(
        _tc_final_body,
        grid=(GRID,),
        in_specs=[rspec, rspec, rspec, dspec, bspec, rspec,
                  wspec, wspec, bspec,
                  pl.BlockSpec((D, 1), lambda i: (0, 0)),
                  pl.BlockSpec((1, 1), lambda i: (0, 0))],
        out_specs=[dspec],
        out_shape=[jax.ShapeDtypeStruct((NPAD, 1), F32)],
    )(sa, sb, t2, dinv, b2, h_in, Wm1a, Wm1b, bm1, Wm2, bm2)[0]


def kernel(x, edge_index, edge_weight, W_in, b_in, W1, b1, W2, b2,
           Wm1, bm1, Wm2, bm2):
    x_p = jnp.pad(x, ((0, NPAD - N), (0, 0)))
    row = edge_index[0]
    col = edge_index[1]
    pad_e = EPAD - E
    row_p = jnp.pad(row, (0, pad_e)).reshape(NW * BPW, BLK)
    col_p = jnp.pad(col, (0, pad_e)).reshape(NW * BPW, BLK)
    ew_p = jnp.pad(edge_weight, (0, pad_e)).reshape(NW * BPW, BLK)

    b_in2 = b_in.reshape(1, D)
    b1_2 = b1.reshape(1, D)
    b2_2 = b2.reshape(1, D)
    bm1_2 = bm1.reshape(1, D)
    bm2_2 = bm2.reshape(1, 1)
    Wm1a = Wm1[:D]
    Wm1b = Wm1[D:]

    deg_parts = _sc_deg(col_p, ew_p)
    h_in, t1, g1, dinv = _tc_lin1(x_p, W_in, b_in2, W1, deg_parts)

    s1 = _sc_conv(g1, row_p, col_p, ew_p)
    t2, g2 = _tc_comb(s1[0], s1[1], t1, dinv, b1_2, W2)

    s2 = _sc_conv(g2, row_p, col_p, ew_p)
    logits = _tc_final(s2[0], s2[1], t2, dinv, b2_2, h_in,
                       Wm1a, Wm1b, bm1_2, Wm2, bm2_2)

    return logits[:N, 0]


# speedup vs baseline: 5.8899x; 5.8899x over previous
import functools

import jax
import jax.numpy as jnp
from jax import lax
from jax.experimental import pallas as pl
from jax.experimental.pallas import tpu as pltpu
from jax.experimental.pallas import tpu_sc as plsc

NC = 2
NS = 16
NW = NC * NS
LANES = 16

N = 10000
NPAD = 10240
D = 128
BLK = 128
E = 320000
BPW = (-(-E // (NW * BLK)) + 7) // 8 * 8
EPAD = NW * BPW * BLK

ROWS_PER_TILE = NPAD // NS
F32 = jnp.float32


def _mesh():
    return plsc.VectorSubcoreMesh(
        core_axis_name="c", subcore_axis_name="s", num_cores=NC, num_subcores=NS)


def _bcast_lane(wv, l):
    idx = jnp.full((LANES, 1), l, jnp.int32)
    dn = lax.GatherDimensionNumbers(
        offset_dims=(), collapsed_slice_dims=(0,), start_index_map=(0,))
    return lax.gather(wv, idx, dn, slice_sizes=(1,),
                      mode=lax.GatherScatterMode.PROMISE_IN_BOUNDS)


DEGW = LANES


def _sc_deg_body(col_hbm, ew16_hbm, z_hbm, out_hbm, col_v, buf, deg_sh):
    cid = lax.axis_index("c")
    sid = lax.axis_index("s")
    w = cid * NS + sid
    pltpu.sync_copy(col_hbm.at[pl.ds(w * BPW, BPW)], col_v)
    pltpu.sync_copy(z_hbm, deg_sh.at[pl.ds(sid * ROWS_PER_TILE, ROWS_PER_TILE)])
    plsc.subcore_barrier()

    @pl.loop(0, BPW)
    def _(j):
        pltpu.sync_copy(ew16_hbm.at[pl.ds((w * BPW + j) * BLK, BLK)], buf)
        pltpu.sync_copy(buf, deg_sh.at[col_v.at[j]], add=True)

    plsc.subcore_barrier()
    pltpu.sync_copy(deg_sh.at[pl.ds(sid * ROWS_PER_TILE, ROWS_PER_TILE)],
                    out_hbm.at[cid, pl.ds(sid * ROWS_PER_TILE, ROWS_PER_TILE)])


def _sc_deg(col_p, ew16, z16):
    k = pl.kernel(
        _sc_deg_body,
        out_type=jax.ShapeDtypeStruct((NC, NPAD, DEGW), F32),
        mesh=_mesh(),
        scratch_types=[
            pltpu.VMEM((BPW, BLK), jnp.int32),
            pltpu.VMEM((BLK, DEGW), F32),
            pltpu.VMEM_SHARED((NPAD, DEGW), F32),
        ],
    )
    return k(col_p, ew16, z16)


def _sc_conv_body(g_hbm, row_hbm, col_hbm, ew_hbm, out_hbm,
                  row_v, col_v, ew_v, gbuf, acc):
    cid = lax.axis_index("c")
    sid = lax.axis_index("s")
    w = cid * NS + sid

    pltpu.sync_copy(row_hbm.at[pl.ds(w * BPW, BPW)], row_v)
    pltpu.sync_copy(col_hbm.at[pl.ds(w * BPW, BPW)], col_v)
    pltpu.sync_copy(ew_hbm.at[pl.ds(w * BPW, BPW)], ew_v)

    zero = jnp.zeros((LANES,), F32)

    @pl.loop(0, BLK)
    def _(r):
        for c8 in range(D // LANES):
            gbuf[r, pl.ds(c8 * LANES, LANES)] = zero

    for k in range(ROWS_PER_TILE // BLK):
        pltpu.sync_copy(gbuf, acc.at[pl.ds(sid * ROWS_PER_TILE + k * BLK, BLK)])
    plsc.subcore_barrier()

    @pl.loop(0, BPW)
    def _(j):
        pltpu.sync_copy(g_hbm.at[row_v.at[j]], gbuf)

        @pl.loop(0, BLK // LANES)
        def _(gi):
            wv = ew_v[j, pl.ds(gi * LANES, LANES)]
            for l in range(LANES):
                bc = _bcast_lane(wv, l)
                e = gi * LANES + l
                for d2 in range(D // LANES):
                    sl = pl.ds(d2 * LANES, LANES)
                    gbuf[e, sl] = gbuf[e, sl] * bc

        pltpu.sync_copy(gbuf, acc.at[col_v.at[j]], add=True)

    plsc.subcore_barrier()
    pltpu.sync_copy(acc.at[pl.ds(sid * ROWS_PER_TILE, ROWS_PER_TILE)],
                    out_hbm.at[cid, pl.ds(sid * ROWS_PER_TILE, ROWS_PER_TILE)])


def _sc_conv(g, row_p, col_p, ew_p):
    k = pl.kernel(
        _sc_conv_body,
        out_type=jax.ShapeDtypeStruct((NC, NPAD, D), F32),
        mesh=_mesh(),
        scratch_types=[
            pltpu.VMEM((BPW, BLK), jnp.int32),
            pltpu.VMEM((BPW, BLK), jnp.int32),
            pltpu.VMEM((BPW, BLK), F32),
            pltpu.VMEM((BLK, D), F32),
            pltpu.VMEM_SHARED((NPAD, D), F32),
        ],
    )
    return k(g, row_p, col_p, ew_p)


RB = 1024
GRID = NPAD // RB


def _tc_lin1_body(x_ref, win_ref, bin_ref, w1_ref, degp_ref,
                  hin_ref, t1_ref, g1_ref, dinv_ref):
    xin = x_ref[...]
    hin = jnp.maximum(xin @ win_ref[...] + bin_ref[...], 0.0)
    deg = degp_ref[0] + degp_ref[1]
    dinv = lax.rsqrt(deg[:, 0:1] + 1.0)
    t1 = hin @ w1_ref[...]
    hin_ref[...] = hin
    t1_ref[...] = t1
    g1_ref[...] = t1 * dinv
    dinv_ref[...] = dinv


def _tc_lin1(x_p, W_in, b_in, W1, deg_parts):
    wspec = pl.BlockSpec((D, D), lambda i: (0, 0))
    bspec = pl.BlockSpec((1, D), lambda i: (0, 0))
    rspec = pl.BlockSpec((RB, D), lambda i: (i, 0))
    return pl.pallas_call(
        _tc_lin1_body,
        grid=(GRID,),
        in_specs=[rspec, wspec, bspec, wspec,
                  pl.BlockSpec((NC, RB, DEGW), lambda i: (0, i, 0))],
        out_specs=[rspec, rspec, rspec, pl.BlockSpec((RB, 1), lambda i: (i, 0))],
        out_shape=[
            jax.ShapeDtypeStruct((NPAD, D), F32),
            jax.ShapeDtypeStruct((NPAD, D), F32),
            jax.ShapeDtypeStruct((NPAD, D), F32),
            jax.ShapeDtypeStruct((NPAD, 1), F32),
        ],
    )(x_p, W_in, b_in, W1, deg_parts)


def _tc_comb_body(sa_ref, sb_ref, t_ref, dinv_ref, b_ref, w2_ref,
                  t2_ref, g2_ref):
    dinv = dinv_ref[...]
    t = t_ref[...]
    h = jnp.maximum(dinv * (sa_ref[...] + sb_ref[...]) + t * (dinv * dinv)
                    + b_ref[...], 0.0)
    t2 = h @ w2_ref[...]
    t2_ref[...] = t2
    g2_ref[...] = t2 * dinv


def _tc_comb(sa, sb, t1, dinv, b1, W2):
    wspec = pl.BlockSpec((D, D), lambda i: (0, 0))
    bspec = pl.BlockSpec((1, D), lambda i: (0, 0))
    rspec = pl.BlockSpec((RB, D), lambda i: (i, 0))
    dspec = pl.BlockSpec((RB, 1), lambda i: (i, 0))
    return pl.pallas_call(
        _tc_comb_body,
        grid=(GRID,),
        in_specs=[rspec, rspec, rspec, dspec, bspec, wspec],
        out_specs=[rspec, rspec],
        out_shape=[
            jax.ShapeDtypeStruct((NPAD, D), F32),
            jax.ShapeDtypeStruct((NPAD, D), F32),
        ],
    )(sa, sb, t1, dinv, b1, W2)


def _tc_final_body(sa_ref, sb_ref, t2_ref, dinv_ref, b2_ref, hin_ref,
                   wm1a_ref, wm1b_ref, bm1_ref, wm2_ref, bm2_ref, out_ref):
    dinv = dinv_ref[...]
    t2 = t2_ref[...]
    h2 = jnp.maximum(dinv * (sa_ref[...] + sb_ref[...]) + t2 * (dinv * dinv)
                     + b2_ref[...], 0.0)
    hidden = jnp.maximum(
        hin_ref[...] @ wm1a_ref[...] + h2 @ wm1b_ref[...] + bm1_ref[...], 0.0)
    out_ref[...] = hidden @ wm2_ref[...] + bm2_ref[...]


def _tc_final(sa, sb, t2, dinv, b2, h_in, Wm1a, Wm1b, bm1, Wm2, bm2):
    wspec = pl.BlockSpec((D, D), lambda i: (0, 0))
    bspec = pl.BlockSpec((1, D), lambda i: (0, 0))
    rspec = pl.BlockSpec((RB, D), lambda i: (i, 0))
    dspec = pl.BlockSpec((RB, 1), lambda i: (i, 0))
    return pl.pallas_call(
        _tc_final_body,
        grid=(GRID,),
        in_specs=[rspec, rspec, rspec, dspec, bspec, rspec,
                  wspec, wspec, bspec,
                  pl.BlockSpec((D, 1), lambda i: (0, 0)),
                  pl.BlockSpec((1, 1), lambda i: (0, 0))],
        out_specs=[dspec],
        out_shape=[jax.ShapeDtypeStruct((NPAD, 1), F32)],
    )(sa, sb, t2, dinv, b2, h_in, Wm1a, Wm1b, bm1, Wm2, bm2)[0]


def kernel(x, edge_index, edge_weight, W_in, b_in, W1, b1, W2, b2,
           Wm1, bm1, Wm2, bm2):
    x_p = jnp.pad(x, ((0, NPAD - N), (0, 0)))
    row = edge_index[0]
    col = edge_index[1]
    pad_e = EPAD - E
    row_p = jnp.pad(row, (0, pad_e)).reshape(NW * BPW, BLK)
    col_p = jnp.pad(col, (0, pad_e)).reshape(NW * BPW, BLK)
    ew_p = jnp.pad(edge_weight, (0, pad_e)).reshape(NW * BPW, BLK)

    b_in2 = b_in.reshape(1, D)
    b1_2 = b1.reshape(1, D)
    b2_2 = b2.reshape(1, D)
    bm1_2 = bm1.reshape(1, D)
    bm2_2 = bm2.reshape(1, 1)
    Wm1a = Wm1[:D]
    Wm1b = Wm1[D:]

    ew16 = jnp.pad(edge_weight[:, None], ((0, pad_e), (0, DEGW - 1)))
    z16 = jnp.zeros((ROWS_PER_TILE, DEGW), F32)
    deg_parts = _sc_deg(col_p, ew16, z16)
    h_in, t1, g1, dinv = _tc_lin1(x_p, W_in, b_in2, W1, deg_parts)

    s1 = _sc_conv(g1, row_p, col_p, ew_p)
    t2, g2 = _tc_comb(s1[0], s1[1], t1, dinv, b1_2, W2)

    s2 = _sc_conv(g2, row_p, col_p, ew_p)
    logits = _tc_final(s2[0], s2[1], t2, dinv, b2_2, h_in,
                       Wm1a, Wm1b, bm1_2, Wm2, bm2_2)

    return logits[:N, 0]


# speedup vs baseline: 6.7014x; 1.1378x over previous
import functools

import jax
import jax.numpy as jnp
from jax import lax
from jax.experimental import pallas as pl
from jax.experimental.pallas import tpu as pltpu
from jax.experimental.pallas import tpu_sc as plsc

NC = 2
NS = 16
NW = NC * NS
LANES = 16

N = 10000
NPAD = 10240
D = 128
BLK = 128
E = 320000
BPW = (-(-E // (NW * BLK)) + 7) // 8 * 8
EPAD = NW * BPW * BLK

ROWS_PER_TILE = NPAD // NS
F32 = jnp.float32


def _mesh():
    return plsc.VectorSubcoreMesh(
        core_axis_name="c", subcore_axis_name="s", num_cores=NC, num_subcores=NS)


def _bcast_lane(wv, l):
    idx = jnp.full((LANES, 1), l, jnp.int32)
    dn = lax.GatherDimensionNumbers(
        offset_dims=(), collapsed_slice_dims=(0,), start_index_map=(0,))
    return lax.gather(wv, idx, dn, slice_sizes=(1,),
                      mode=lax.GatherScatterMode.PROMISE_IN_BOUNDS)


DEGW = LANES


def _sc_deg_body(col_hbm, ew16_hbm, z_hbm, out_hbm, col_v, buf, deg_sh):
    cid = lax.axis_index("c")
    sid = lax.axis_index("s")
    w = cid * NS + sid
    pltpu.sync_copy(col_hbm.at[pl.ds(w * BPW, BPW)], col_v)
    pltpu.sync_copy(z_hbm, deg_sh.at[pl.ds(sid * ROWS_PER_TILE, ROWS_PER_TILE)])
    plsc.subcore_barrier()

    @pl.loop(0, BPW)
    def _(j):
        pltpu.sync_copy(ew16_hbm.at[pl.ds((w * BPW + j) * BLK, BLK)], buf)
        pltpu.sync_copy(buf, deg_sh.at[col_v.at[j]], add=True)

    plsc.subcore_barrier()
    pltpu.sync_copy(deg_sh.at[pl.ds(sid * ROWS_PER_TILE, ROWS_PER_TILE)],
                    out_hbm.at[cid, pl.ds(sid * ROWS_PER_TILE, ROWS_PER_TILE)])


def _sc_deg(col_p, ew16, z16):
    k = pl.kernel(
        _sc_deg_body,
        out_type=jax.ShapeDtypeStruct((NC, NPAD, DEGW), F32),
        mesh=_mesh(),
        scratch_types=[
            pltpu.VMEM((BPW, BLK), jnp.int32),
            pltpu.VMEM((BLK, DEGW), F32),
            pltpu.VMEM_SHARED((NPAD, DEGW), F32),
        ],
    )
    return k(col_p, ew16, z16)


BLKR = 64
BPWR = EPAD // (NW * BLKR)
NH = BPWR // 2
NRB = 4


def _sc_conv_body(g_hbm, row_hbm, col_hbm, ew_hbm, out_hbm,
                  row_v, col_v, ew_v, gb0, gb1, gb2, gb3, acc,
                  gsem0, gsem1, gsem2, gsem3, ssem0, ssem1, ssem2, ssem3):
    cid = lax.axis_index("c")
    sid = lax.axis_index("s")
    w = cid * NS + sid
    gbs = (gb0, gb1, gb2, gb3)
    gsems = (gsem0, gsem1, gsem2, gsem3)
    ssems = (ssem0, ssem1, ssem2, ssem3)

    zero = jnp.zeros((LANES,), F32)

    @pl.loop(0, BLKR)
    def _(r):
        for c8 in range(D // LANES):
            gb0[r, pl.ds(c8 * LANES, LANES)] = zero

    for k in range(ROWS_PER_TILE // BLKR):
        pltpu.sync_copy(gb0, acc.at[pl.ds(sid * ROWS_PER_TILE + k * BLKR, BLKR)])
    plsc.subcore_barrier()

    def _scale(b, q, off):
        @pl.loop(0, BLKR // LANES)
        def _(gi):
            wv = ew_v[q, pl.ds(off + gi * LANES, LANES)]
            for l in range(LANES):
                bc = _bcast_lane(wv, l)
                e = gi * LANES + l
                for d2 in range(D // LANES):
                    sl = pl.ds(d2 * LANES, LANES)
                    gbs[b][e, sl] = gbs[b][e, sl] * bc

    for h in range(2):
        base = w * BPW + h * (BPW // 2)
        pltpu.sync_copy(row_hbm.at[pl.ds(base, BPW // 2)], row_v)
        pltpu.sync_copy(col_hbm.at[pl.ds(base, BPW // 2)], col_v)
        pltpu.sync_copy(ew_hbm.at[pl.ds(base, BPW // 2)], ew_v)

        for b in range(3):
            pltpu.async_copy(
                g_hbm.at[row_v.at[b // 2, pl.ds((b % 2) * BLKR, BLKR)]],
                gbs[b], gsems[b])

        @pl.loop(0, NH // NRB)
        def _(i):
            for b in range(NRB):
                q = 2 * i + b // 2
                off = (b % 2) * BLKR
                q3 = 2 * i + (b + 3) // 2
                off3 = ((b + 3) % 2) * BLKR
                b3 = (b + 3) % NRB
                pltpu.make_async_copy(
                    g_hbm.at[pl.ds(0, BLKR)], gbs[b], gsems[b]).wait()
                _scale(b, q, off)
                pltpu.async_copy(
                    gbs[b], acc.at[col_v.at[q, pl.ds(off, BLKR)]], ssems[b],
                    add=True)
                if b == 0:
                    @pl.when(i >= 1)
                    def _():
                        pltpu.make_async_copy(
                            g_hbm.at[pl.ds(0, BLKR)], gbs[b3], ssems[b3]).wait()
                    pltpu.async_copy(
                        g_hbm.at[row_v.at[q3, pl.ds(off3, BLKR)]], gbs[b3],
                        gsems[b3])
                else:
                    pltpu.make_async_copy(
                        g_hbm.at[pl.ds(0, BLKR)], gbs[b3], ssems[b3]).wait()

                    @pl.when(i <= NH // NRB - 2)
                    def _():
                        pltpu.async_copy(
                            g_hbm.at[row_v.at[q3, pl.ds(off3, BLKR)]], gbs[b3],
                            gsems[b3])

        pltpu.make_async_copy(g_hbm.at[pl.ds(0, BLKR)], gbs[3], ssems[3]).wait()

    plsc.subcore_barrier()
    pltpu.sync_copy(acc.at[pl.ds(sid * ROWS_PER_TILE, ROWS_PER_TILE)],
                    out_hbm.at[cid, pl.ds(sid * ROWS_PER_TILE, ROWS_PER_TILE)])


def _sc_conv(g, row_r, col_r, ew_r):
    k = pl.kernel(
        _sc_conv_body,
        out_type=jax.ShapeDtypeStruct((NC, NPAD, D), F32),
        mesh=_mesh(),
        scratch_types=[
            pltpu.VMEM((BPW // 2, BLK), jnp.int32),
            pltpu.VMEM((BPW // 2, BLK), jnp.int32),
            pltpu.VMEM((BPW // 2, BLK), F32),
            pltpu.VMEM((BLKR, D), F32),
            pltpu.VMEM((BLKR, D), F32),
            pltpu.VMEM((BLKR, D), F32),
            pltpu.VMEM((BLKR, D), F32),
            pltpu.VMEM_SHARED((NPAD, D), F32),
            pltpu.SemaphoreType.DMA,
            pltpu.SemaphoreType.DMA,
            pltpu.SemaphoreType.DMA,
            pltpu.SemaphoreType.DMA,
            pltpu.SemaphoreType.DMA,
            pltpu.SemaphoreType.DMA,
            pltpu.SemaphoreType.DMA,
            pltpu.SemaphoreType.DMA,
        ],
    )
    return k(g, row_r, col_r, ew_r)


RB = 1024
GRID = NPAD // RB


def _tc_lin1_body(x_ref, win_ref, bin_ref, w1_ref, degp_ref,
                  hin_ref, t1_ref, g1_ref, dinv_ref):
    xin = x_ref[...]
    hin = jnp.maximum(xin @ win_ref[...] + bin_ref[...], 0.0)
    deg = degp_ref[0] + degp_ref[1]
    dinv = lax.rsqrt(deg[:, 0:1] + 1.0)
    t1 = hin @ w1_ref[...]
    hin_ref[...] = hin
    t1_ref[...] = t1
    g1_ref[...] = t1 * dinv
    dinv_ref[...] = dinv


def _tc_lin1(x_p, W_in, b_in, W1, deg_parts):
    wspec = pl.BlockSpec((D, D), lambda i: (0, 0))
    bspec = pl.BlockSpec((1, D), lambda i: (0, 0))
    rspec = pl.BlockSpec((RB, D), lambda i: (i, 0))
    return pl.pallas_call(
        _tc_lin1_body,
        grid=(GRID,),
        in_specs=[rspec, wspec, bspec, wspec,
                  pl.BlockSpec((NC, RB, DEGW), lambda i: (0, i, 0))],
        out_specs=[rspec, rspec, rspec, pl.BlockSpec((RB, 1), lambda i: (i, 0))],
        out_shape=[
            jax.ShapeDtypeStruct((NPAD, D), F32),
            jax.ShapeDtypeStruct((NPAD, D), F32),
            jax.ShapeDtypeStruct((NPAD, D), F32),
            jax.ShapeDtypeStruct((NPAD, 1), F32),
        ],
    )(x_p, W_in, b_in, W1, deg_parts)


def _tc_comb_body(sa_ref, sb_ref, t_ref, dinv_ref, b_ref, w2_ref,
                  t2_ref, g2_ref):
    dinv = dinv_ref[...]
    t = t_ref[...]
    h = jnp.maximum(dinv * (sa_ref[...] + sb_ref[...]) + t * (dinv * dinv)
                    + b_ref[...], 0.0)
    t2 = h @ w2_ref[...]
    t2_ref[...] = t2
    g2_ref[...] = t2 * dinv


def _tc_comb(sa, sb, t1, dinv, b1, W2):
    wspec = pl.BlockSpec((D, D), lambda i: (0, 0))
    bspec = pl.BlockSpec((1, D), lambda i: (0, 0))
    rspec = pl.BlockSpec((RB, D), lambda i: (i, 0))
    dspec = pl.BlockSpec((RB, 1), lambda i: (i, 0))
    return pl.pallas_call(
        _tc_comb_body,
        grid=(GRID,),
        in_specs=[rspec, rspec, rspec, dspec, bspec, wspec],
        out_specs=[rspec, rspec],
        out_shape=[
            jax.ShapeDtypeStruct((NPAD, D), F32),
            jax.ShapeDtypeStruct((NPAD, D), F32),
        ],
    )(sa, sb, t1, dinv, b1, W2)


def _tc_final_body(sa_ref, sb_ref, t2_ref, dinv_ref, b2_ref, hin_ref,
                   wm1a_ref, wm1b_ref, bm1_ref, wm2_ref, bm2_ref, out_ref):
    dinv = dinv_ref[...]
    t2 = t2_ref[...]
    h2 = jnp.maximum(dinv * (sa_ref[...] + sb_ref[...]) + t2 * (dinv * dinv)
                     + b2_ref[...], 0.0)
    hidden = jnp.maximum(
        hin_ref[...] @ wm1a_ref[...] + h2 @ wm1b_ref[...] + bm1_ref[...], 0.0)
    out_ref[...] = hidden @ wm2_ref[...] + bm2_ref[...]


def _tc_final(sa, sb, t2, dinv, b2, h_in, Wm1a, Wm1b, bm1, Wm2, bm2):
    wspec = pl.BlockSpec((D, D), lambda i: (0, 0))
    bspec = pl.BlockSpec((1, D), lambda i: (0, 0))
    rspec = pl.BlockSpec((RB, D), lambda i: (i, 0))
    dspec = pl.BlockSpec((RB, 1), lambda i: (i, 0))
    return pl.pallas_call(
        _tc_final_body,
        grid=(GRID,),
        in_specs=[rspec, rspec, rspec, dspec, bspec, rspec,
                  wspec, wspec, bspec,
                  pl.BlockSpec((D, 1), lambda i: (0, 0)),
                  pl.BlockSpec((1, 1), lambda i: (0, 0))],
        out_specs=[dspec],
        out_shape=[jax.ShapeDtypeStruct((NPAD, 1), F32)],
    )(sa, sb, t2, dinv, b2, h_in, Wm1a, Wm1b, bm1, Wm2, bm2)[0]


def kernel(x, edge_index, edge_weight, W_in, b_in, W1, b1, W2, b2,
           Wm1, bm1, Wm2, bm2):
    x_p = jnp.pad(x, ((0, NPAD - N), (0, 0)))
    row = edge_index[0]
    col = edge_index[1]
    pad_e = EPAD - E
    row_p = jnp.pad(row, (0, pad_e)).reshape(NW * BPW, BLK)
    col_p = jnp.pad(col, (0, pad_e)).reshape(NW * BPW, BLK)
    ew_p = jnp.pad(edge_weight, (0, pad_e)).reshape(NW * BPW, BLK)

    b_in2 = b_in.reshape(1, D)
    b1_2 = b1.reshape(1, D)
    b2_2 = b2.reshape(1, D)
    bm1_2 = bm1.reshape(1, D)
    bm2_2 = bm2.reshape(1, 1)
    Wm1a = Wm1[:D]
    Wm1b = Wm1[D:]

    ew16 = jnp.pad(edge_weight[:, None], ((0, pad_e), (0, DEGW - 1)))
    z16 = jnp.zeros((ROWS_PER_TILE, DEGW), F32)
    deg_parts = _sc_deg(col_p, ew16, z16)
    h_in, t1, g1, dinv = _tc_lin1(x_p, W_in, b_in2, W1, deg_parts)

    s1 = _sc_conv(g1, row_p, col_p, ew_p)
    t2, g2 = _tc_comb(s1[0], s1[1], t1, dinv, b1_2, W2)

    s2 = _sc_conv(g2, row_p, col_p, ew_p)
    logits = _tc_final(s2[0], s2[1], t2, dinv, b2_2, h_in,
                       Wm1a, Wm1b, bm1_2, Wm2, bm2_2)

    return logits[:N, 0]
